# baseline (speedup 1.0000x reference)
import functools

import jax
import jax.numpy as jnp
from jax import lax
from jax.experimental import pallas as pl
from jax.experimental.pallas import tpu as pltpu
from jax.experimental.pallas import tpu_sc as plsc

CACHE = 1_000_000
S = 16384
NC = 2
NS = 16
HALF = CACHE // NC
ROWS = 8
LANES = 128
PAD = NC * NS * 16
OUT = CACHE + PAD

CP = 31256
CP_LAST = HALF - (NS - 1) * CP


def _sc_kernel(idx_hbm, val_hbm, cache_hbm, out_hbm, idx_v, val_v, cp_v):
    c = lax.axis_index("c")
    s = lax.axis_index("s")
    lo = c * HALF

    @pl.when(s < NS - 1)
    def _():
        start = lo + s * CP
        pltpu.sync_copy(cache_hbm.at[pl.ds(start, CP)], cp_v)
        pltpu.sync_copy(cp_v, out_hbm.at[pl.ds(start, CP)])

    @pl.when(s == NS - 1)
    def _():
        start = lo + (NS - 1) * CP
        pltpu.sync_copy(cache_hbm.at[pl.ds(start, CP_LAST)],
                        cp_v.at[pl.ds(0, CP_LAST)])
        pltpu.sync_copy(cp_v.at[pl.ds(0, CP_LAST)],
                        out_hbm.at[pl.ds(start, CP_LAST)])

    pltpu.sync_copy(idx_hbm.at[s], idx_v)
    pltpu.sync_copy(val_hbm.at[s], val_v)

    pad_base = CACHE + (s * NC + c) * 16
    lane = lax.iota(jnp.int32, 16)
    for r in range(ROWS):
        for j in range(LANES // 16):
            sl = (r, pl.ds(j * 16, 16))
            iv = idx_v[sl]
            owned = (iv >= lo) & (iv < lo + HALF)
            idx_v[sl] = jnp.where(owned, iv, pad_base + lane)

    plsc.subcore_barrier()

    for p in range(NS):
        @pl.when(s == p)
        def _():
            for r in range(ROWS):
                pltpu.sync_copy(val_v.at[r], out_hbm.at[idx_v.at[r]])
        plsc.subcore_barrier()


def kernel(input_pos, values, cache):
    idx3 = input_pos.astype(jnp.int32).reshape(NS, ROWS, LANES)
    val3 = values.reshape(NS, ROWS, LANES)
    mesh = plsc.VectorSubcoreMesh(core_axis_name="c", subcore_axis_name="s",
                                  num_cores=NC, num_subcores=NS)

    run = pl.kernel(
        _sc_kernel,
        out_type=jax.ShapeDtypeStruct((OUT,), jnp.float32),
        mesh=mesh,
        scratch_types=[
            pltpu.VMEM((ROWS, LANES), jnp.int32),
            pltpu.VMEM((ROWS, LANES), jnp.float32),
            pltpu.VMEM((CP,), jnp.float32),
        ],
    )
    out = run(idx3, val3, cache)
    return out[:CACHE]


# speedup vs baseline: 41.9860x; 41.9860x over previous
import dataclasses

import jax
import jax.numpy as jnp
from jax import lax
from jax.experimental import pallas as pl
from jax.experimental.pallas import tpu as pltpu
from jax.experimental.pallas import tpu_sc as plsc

CACHE = 1_000_000
S = 16384
NC = 2
NS = 16
NW = NC * NS
GROUPS = S // 16
SENT = 2**30

CPW = 31256
B_LAST = (NW - 1) * CPW
N_LAST = CACHE - B_LAST


def _tc_dedup(idx_ref, out_ref):
    x = idx_ref[...]
    lane = lax.broadcasted_iota(jnp.int32, (128, 128), 1) % 16
    dup = jnp.zeros((128, 128), dtype=jnp.bool_)
    for sh in range(1, 16):
        y = pltpu.roll(x, 128 - sh, 1)
        dup = dup | ((y == x) & (lane < 16 - sh))
    out_ref[...] = jnp.where(dup, SENT, x)


def _sc_scatter(idx_hbm, val_hbm, cache_hbm, out_hbm, idx_v, val_v, buf):
    c = lax.axis_index("c")
    s = lax.axis_index("s")
    w = s * NC + c
    b = w * CPW
    n = jnp.where(w == NW - 1, N_LAST, CPW)

    pltpu.sync_copy(idx_hbm, idx_v)
    pltpu.sync_copy(val_hbm, val_v)

    @pl.when(w < NW - 1)
    def _():
        pltpu.sync_copy(cache_hbm.at[pl.ds(b, CPW)], buf)

    @pl.when(w == NW - 1)
    def _():
        pltpu.sync_copy(cache_hbm.at[pl.ds(B_LAST, N_LAST)],
                        buf.at[pl.ds(0, N_LAST)])

    @pl.loop(0, GROUPS)
    def _(g):
        sl = pl.ds(g * 16, 16)
        loc = idx_v[sl] - b
        inr = (loc >= 0) & (loc < n)
        loc = jnp.where(inr, loc, 0)
        plsc.store_scatter(buf, [loc], val_v[sl], mask=inr)

    @pl.when(w < NW - 1)
    def _():
        pltpu.sync_copy(buf, out_hbm.at[pl.ds(b, CPW)])

    @pl.when(w == NW - 1)
    def _():
        pltpu.sync_copy(buf.at[pl.ds(0, N_LAST)],
                        out_hbm.at[pl.ds(B_LAST, N_LAST)])


def kernel(input_pos, values, cache):
    idx2 = input_pos.astype(jnp.int32).reshape(128, 128)

    idx_d = pl.pallas_call(
        _tc_dedup,
        out_shape=jax.ShapeDtypeStruct((128, 128), jnp.int32),
    )(idx2)
    idx_flat = idx_d.reshape(S)

    mesh = plsc.VectorSubcoreMesh(core_axis_name="c", subcore_axis_name="s",
                                  num_cores=NC, num_subcores=NS)
    cp = pltpu.CompilerParams()
    if "needs_layout_passes" in pltpu.CompilerParams.__dataclass_fields__:
        cp = dataclasses.replace(cp, needs_layout_passes=False)
    run = pl.kernel(
        _sc_scatter,
        out_type=jax.ShapeDtypeStruct((CACHE,), jnp.float32),
        mesh=mesh,
        scratch_types=[
            pltpu.VMEM((S,), jnp.int32),
            pltpu.VMEM((S,), jnp.float32),
            pltpu.VMEM((CPW,), jnp.float32),
        ],
        compiler_params=cp,
    )
    return run(idx_flat, values, cache)


# speedup vs baseline: 45.3610x; 1.0804x over previous
import dataclasses

import jax
import jax.numpy as jnp
from jax import lax
from jax.experimental import pallas as pl
from jax.experimental.pallas import tpu as pltpu
from jax.experimental.pallas import tpu_sc as plsc

CACHE = 1_000_000
S = 16384
NC = 2
NS = 16
NW = NC * NS
GROUPS = S // 16
SENT = 2**30

CPW = 31256
B_LAST = (NW - 1) * CPW
N_LAST = CACHE - B_LAST


def _tc_dedup(idx_ref, out_ref):
    x = idx_ref[...]
    lane = lax.broadcasted_iota(jnp.int32, (128, 128), 1) % 16
    dup = jnp.zeros((128, 128), dtype=jnp.bool_)
    for sh in range(1, 16):
        y = pltpu.roll(x, 128 - sh, 1)
        dup = dup | ((y == x) & (lane < 16 - sh))
    out_ref[...] = jnp.where(dup, SENT, x)


def _sc_scatter(idx_hbm, val_hbm, cache_hbm, out_hbm, idx_v, val_v, buf,
                sem_i, sem_v, sem_b):
    c = lax.axis_index("c")
    s = lax.axis_index("s")
    w = s * NC + c
    b = w * CPW
    n = jnp.where(w == NW - 1, N_LAST, CPW).astype(jnp.uint32)

    pltpu.async_copy(idx_hbm, idx_v, sem_i)
    pltpu.async_copy(val_hbm, val_v, sem_v)

    @pl.when(w < NW - 1)
    def _():
        pltpu.async_copy(cache_hbm.at[pl.ds(b, CPW)], buf, sem_b)

    @pl.when(w == NW - 1)
    def _():
        pltpu.async_copy(cache_hbm.at[pl.ds(B_LAST, N_LAST)],
                         buf.at[pl.ds(0, N_LAST)], sem_b)

    pltpu.make_async_copy(idx_hbm, idx_v, sem_i).wait()
    pltpu.make_async_copy(val_hbm, val_v, sem_v).wait()

    @pl.when(w < NW - 1)
    def _():
        pltpu.make_async_copy(cache_hbm.at[pl.ds(b, CPW)], buf, sem_b).wait()

    @pl.when(w == NW - 1)
    def _():
        pltpu.make_async_copy(cache_hbm.at[pl.ds(B_LAST, N_LAST)],
                              buf.at[pl.ds(0, N_LAST)], sem_b).wait()

    @pl.loop(0, GROUPS, step=8)
    def _(g0):
        for t in range(8):
            sl = pl.ds((g0 + t) * 16, 16)
            loc = idx_v[sl] - b
            inr = plsc.bitcast(loc, jnp.uint32) < n
            plsc.store_scatter(buf, [loc], val_v[sl], mask=inr)

    @pl.when(w < NW - 1)
    def _():
        pltpu.sync_copy(buf, out_hbm.at[pl.ds(b, CPW)])

    @pl.when(w == NW - 1)
    def _():
        pltpu.sync_copy(buf.at[pl.ds(0, N_LAST)],
                        out_hbm.at[pl.ds(B_LAST, N_LAST)])


def kernel(input_pos, values, cache):
    idx2 = input_pos.astype(jnp.int32).reshape(128, 128)

    idx_d = pl.pallas_call(
        _tc_dedup,
        out_shape=jax.ShapeDtypeStruct((128, 128), jnp.int32),
    )(idx2)
    idx_flat = idx_d.reshape(S)

    mesh = plsc.VectorSubcoreMesh(core_axis_name="c", subcore_axis_name="s",
                                  num_cores=NC, num_subcores=NS)
    cp = pltpu.CompilerParams()
    if "needs_layout_passes" in pltpu.CompilerParams.__dataclass_fields__:
        cp = dataclasses.replace(cp, needs_layout_passes=False)
    run = pl.kernel(
        _sc_scatter,
        out_type=jax.ShapeDtypeStruct((CACHE,), jnp.float32),
        mesh=mesh,
        scratch_types=[
            pltpu.VMEM((S,), jnp.int32),
            pltpu.VMEM((S,), jnp.float32),
            pltpu.VMEM((CPW,), jnp.float32),
            pltpu.SemaphoreType.DMA,
            pltpu.SemaphoreType.DMA,
            pltpu.SemaphoreType.DMA,
        ],
        compiler_params=cp,
    )
    return run(idx_flat, values, cache)
